# baseline (speedup 1.0000x reference)
import jax
import jax.numpy as jnp
from jax import lax
from jax.experimental import pallas as pl
from jax.experimental.pallas import tpu as pltpu
from jax.experimental.pallas import tpu_sc as plsc

_N = 10000
_NP = 10240
_D = 128
_DH = 64
_G = 64
_NC = 2
_NS = 16
_W = 128
_NWIN = 160
_EPT = _W * _NWIN
_EPAD = _EPT * _NS
_RPT = _NP // _NS
_B = 10240
_GRID = _NP // _B

_HIGH = jax.lax.Precision.HIGHEST


_NBUF = 4


def _sc_body(h_hbm, src_hbm, dst_hbm, out_hbm,
             sidx, didx, bufs, acc, gsem, ssem):
    c = lax.axis_index("c")
    s = lax.axis_index("s")
    table = h_hbm.at[c]

    pltpu.make_async_copy(src_hbm.at[s], sidx, gsem.at[0]).start()
    pltpu.make_async_copy(dst_hbm.at[s], didx, gsem.at[1]).start()

    @pl.loop(0, _W)
    def _zero_row(r):
        for col in range(0, _DH, 16):
            bufs[0, r, pl.ds(col, 16)] = jnp.zeros((16,), jnp.float32)

    for z in range(_RPT // _W):
        pltpu.make_async_copy(bufs.at[0],
                              acc.at[pl.ds(s * _RPT + z * _W, _W)],
                              ssem.at[0]).start()
    for z in range(_RPT // _W):
        pltpu.make_async_copy(bufs.at[0],
                              acc.at[pl.ds(s * _RPT + z * _W, _W)],
                              ssem.at[0]).wait()

    pltpu.make_async_copy(src_hbm.at[s], sidx, gsem.at[0]).wait()
    pltpu.make_async_copy(dst_hbm.at[s], didx, gsem.at[1]).wait()

    for b in range(_NBUF):
        pltpu.make_async_copy(table.at[sidx.at[b]], bufs.at[b],
                              gsem.at[b]).start()

    plsc.subcore_barrier()

    @pl.loop(0, _NWIN, step=_NBUF)
    def _win(j):
        for b in range(_NBUF):
            pltpu.make_async_copy(table.at[sidx.at[j + b]], bufs.at[b],
                                  gsem.at[b]).wait()
            pltpu.make_async_copy(bufs.at[b], acc.at[didx.at[j + b]],
                                  ssem.at[b]).start(add=True)
        for b in range(_NBUF):
            pltpu.make_async_copy(bufs.at[b], acc.at[didx.at[j + b]],
                                  ssem.at[b]).wait()

            @pl.when(j + _NBUF + b < _NWIN)
            def _():
                pltpu.make_async_copy(table.at[sidx.at[j + _NBUF + b]],
                                      bufs.at[b], gsem.at[b]).start()

    plsc.subcore_barrier()

    pltpu.sync_copy(acc.at[pl.ds(s * _RPT, _RPT)],
                    out_hbm.at[c, pl.ds(s * _RPT, _RPT)])


def _sc_aggregate(h2, src3, dst3):
    mesh = plsc.VectorSubcoreMesh(core_axis_name="c", subcore_axis_name="s")
    k = pl.kernel(
        _sc_body,
        out_type=jax.ShapeDtypeStruct((_NC, _NP, _DH), jnp.float32),
        mesh=mesh,
        compiler_params=pltpu.CompilerParams(use_tc_tiling_on_sc=False),
        scratch_types=[
            pltpu.VMEM((_NWIN, _W), jnp.int32),
            pltpu.VMEM((_NWIN, _W), jnp.int32),
            pltpu.VMEM((_NBUF, _W, _DH), jnp.float32),
            pltpu.VMEM_SHARED((_NP, _DH), jnp.float32),
            pltpu.SemaphoreType.DMA((_NBUF,)),
            pltpu.SemaphoreType.DMA((_NBUF,)),
        ],
    )
    return k(h2, src3, dst3)


def _tc_root_body(h_ref, wroot_ref, b_ref, out_ref):
    h = jnp.concatenate([h_ref[0], h_ref[1]], axis=1)
    res = jnp.dot(h, wroot_ref[...], precision=_HIGH,
                  preferred_element_type=jnp.float32) + b_ref[...]
    out_ref[0] = res[:, :_DH]
    out_ref[1] = res[:, _DH:]


def _tc_root(h2, wroot, b):
    return pl.pallas_call(
        _tc_root_body,
        grid=(_GRID,),
        in_specs=[
            pl.BlockSpec((_NC, _B, _DH), lambda i: (0, i, 0)),
            pl.BlockSpec((_D, _D), lambda i: (0, 0)),
            pl.BlockSpec((1, _D), lambda i: (0, 0)),
        ],
        out_specs=pl.BlockSpec((_NC, _B, _DH), lambda i: (0, i, 0)),
        out_shape=jax.ShapeDtypeStruct((_NC, _NP, _DH), jnp.float32),
    )(h2, wroot, b)


def _tc_layer_body(p_ref, r_ref, wrel_ref, out_ref):
    agg = jnp.concatenate([p_ref[0], p_ref[1]], axis=1)
    root = jnp.concatenate([r_ref[0], r_ref[1]], axis=1)
    acc = jnp.dot(agg, wrel_ref[...], precision=_HIGH,
                  preferred_element_type=jnp.float32)
    res = jnp.maximum(acc + root, 0.0)
    out_ref[0] = res[:, :_DH]
    out_ref[1] = res[:, _DH:]


def _tc_layer(p, r, wrel):
    return pl.pallas_call(
        _tc_layer_body,
        grid=(_GRID,),
        in_specs=[
            pl.BlockSpec((_NC, _B, _DH), lambda i: (0, i, 0)),
            pl.BlockSpec((_NC, _B, _DH), lambda i: (0, i, 0)),
            pl.BlockSpec((_D, _D), lambda i: (0, 0)),
        ],
        out_specs=pl.BlockSpec((_NC, _B, _DH), lambda i: (0, i, 0)),
        out_shape=jax.ShapeDtypeStruct((_NC, _NP, _DH), jnp.float32),
    )(p, r, wrel)


def _pool_body(h_ref, batch_ref, wfc1_ref, bfc1_ref, wfc2_ref, bfc2_ref,
               out_ref, pooled, counts):
    i = pl.program_id(0)

    @pl.when(i == 0)
    def _():
        pooled[...] = jnp.zeros_like(pooled)
        counts[...] = jnp.zeros_like(counts)

    h = jnp.concatenate([h_ref[0], h_ref[1]], axis=1)
    ids = lax.broadcasted_iota(jnp.int32, (_G, _B), 0)
    seg = (batch_ref[...] == ids).astype(jnp.float32)
    pooled[...] += jnp.dot(seg, h, precision=_HIGH,
                           preferred_element_type=jnp.float32)
    counts[...] += jnp.broadcast_to(jnp.sum(seg, axis=1)[:, None], (_G, _D))

    @pl.when(i == _GRID - 1)
    def _():
        pm = pooled[...] / jnp.maximum(counts[...], 1.0)
        o1 = jnp.dot(pm, wfc1_ref[...], precision=_HIGH,
                     preferred_element_type=jnp.float32) + bfc1_ref[...]
        out_ref[...] = jnp.dot(o1, wfc2_ref[...], precision=_HIGH,
                               preferred_element_type=jnp.float32) + bfc2_ref[...]


def _pool_mlp(h2, batch2, wfc1, bfc1, wfc2p, bfc2p):
    return pl.pallas_call(
        _pool_body,
        grid=(_GRID,),
        in_specs=[
            pl.BlockSpec((_NC, _B, _DH), lambda i: (0, i, 0)),
            pl.BlockSpec((1, _B), lambda i: (0, i)),
            pl.BlockSpec((_D, _D), lambda i: (0, 0)),
            pl.BlockSpec((1, _D), lambda i: (0, 0)),
            pl.BlockSpec((_D, _D), lambda i: (0, 0)),
            pl.BlockSpec((1, _D), lambda i: (0, 0)),
        ],
        out_specs=pl.BlockSpec((_G, _D), lambda i: (0, 0)),
        out_shape=jax.ShapeDtypeStruct((_G, _D), jnp.float32),
        scratch_shapes=[
            pltpu.VMEM((_G, _D), jnp.float32),
            pltpu.VMEM((_G, _D), jnp.float32),
        ],
    )(h2, batch2, wfc1, bfc1, wfc2p, bfc2p)


def kernel(x, edge_index, batch, W_rel1, b_rel1, W_root1, W_rel2, b_rel2,
           W_root2, W_rel3, b_rel3, W_root3, W_fc1, b_fc1, W_fc2, b_fc2):
    out_dim = W_fc2.shape[1]
    npad = _NP - _N
    epad = _EPAD - edge_index.shape[1]

    xp = jnp.concatenate([x, jnp.zeros((npad, _D), x.dtype)], axis=0)
    x2 = jnp.stack([xp[:, :_DH], xp[:, _DH:]])
    fill = (jnp.arange(epad, dtype=jnp.int32) % npad) + _N
    src3 = jnp.concatenate([edge_index[0], fill]).reshape(_NS, _NWIN, _W)
    dst3 = jnp.concatenate([edge_index[1], fill]).reshape(_NS, _NWIN, _W)
    batch2 = jnp.concatenate(
        [batch, jnp.full((npad,), _G, jnp.int32)]).reshape(1, _NP)

    h2 = x2
    for wrel, brel, wroot in ((W_rel1, b_rel1, W_root1),
                              (W_rel2, b_rel2, W_root2),
                              (W_rel3, b_rel3, W_root3)):
        p = _sc_aggregate(h2, src3, dst3)
        r = _tc_root(h2, wroot, brel.reshape(1, _D))
        h2 = _tc_layer(p, r, wrel)

    wfc2p = jnp.pad(W_fc2, ((0, 0), (0, _D - out_dim)))
    bfc2p = jnp.pad(b_fc2, (0, _D - out_dim)).reshape(1, _D)
    out = _pool_mlp(h2, batch2, W_fc1, b_fc1.reshape(1, _D), wfc2p, bfc2p)
    return out[:, :out_dim]


# speedup vs baseline: 1.0313x; 1.0313x over previous
import jax
import jax.numpy as jnp
from jax import lax
from jax.experimental import pallas as pl
from jax.experimental.pallas import tpu as pltpu
from jax.experimental.pallas import tpu_sc as plsc

_N = 10000
_NP = 10240
_D = 128
_DH = 64
_G = 64
_NC = 2
_NS = 16
_W = 128
_NWIN = 160
_EPT = _W * _NWIN
_EPAD = _EPT * _NS
_RPT = _NP // _NS
_B = 5120
_GRID = _NP // _B

_HIGH = jax.lax.Precision.HIGHEST


_NBUF = 4


def _sc_body(h_hbm, src_hbm, dst_hbm, out_hbm,
             sidx, didx, bufs, acc, gsem, ssem):
    c = lax.axis_index("c")
    s = lax.axis_index("s")
    table = h_hbm.at[c]

    pltpu.make_async_copy(src_hbm.at[s], sidx, gsem.at[0]).start()
    pltpu.make_async_copy(dst_hbm.at[s], didx, gsem.at[1]).start()

    @pl.loop(0, _W)
    def _zero_row(r):
        for col in range(0, _DH, 16):
            bufs[0, r, pl.ds(col, 16)] = jnp.zeros((16,), jnp.float32)

    for z in range(_RPT // _W):
        pltpu.make_async_copy(bufs.at[0],
                              acc.at[pl.ds(s * _RPT + z * _W, _W)],
                              ssem.at[0]).start()
    for z in range(_RPT // _W):
        pltpu.make_async_copy(bufs.at[0],
                              acc.at[pl.ds(s * _RPT + z * _W, _W)],
                              ssem.at[0]).wait()

    pltpu.make_async_copy(src_hbm.at[s], sidx, gsem.at[0]).wait()
    pltpu.make_async_copy(dst_hbm.at[s], didx, gsem.at[1]).wait()

    for b in range(_NBUF):
        pltpu.make_async_copy(table.at[sidx.at[b]], bufs.at[b],
                              gsem.at[b]).start()

    plsc.subcore_barrier()

    @pl.loop(0, _NWIN, step=_NBUF)
    def _win(j):
        for b in range(_NBUF):
            pltpu.make_async_copy(table.at[sidx.at[j + b]], bufs.at[b],
                                  gsem.at[b]).wait()
            pltpu.make_async_copy(bufs.at[b], acc.at[didx.at[j + b]],
                                  ssem.at[b]).start(add=True)
        for b in range(_NBUF):
            pltpu.make_async_copy(bufs.at[b], acc.at[didx.at[j + b]],
                                  ssem.at[b]).wait()

            @pl.when(j + _NBUF + b < _NWIN)
            def _():
                pltpu.make_async_copy(table.at[sidx.at[j + _NBUF + b]],
                                      bufs.at[b], gsem.at[b]).start()

    plsc.subcore_barrier()

    pltpu.sync_copy(acc.at[pl.ds(s * _RPT, _RPT)],
                    out_hbm.at[c, pl.ds(s * _RPT, _RPT)])


def _sc_aggregate(h2, src3, dst3):
    mesh = plsc.VectorSubcoreMesh(core_axis_name="c", subcore_axis_name="s")
    k = pl.kernel(
        _sc_body,
        out_type=jax.ShapeDtypeStruct((_NC, _NP, _DH), jnp.float32),
        mesh=mesh,
        compiler_params=pltpu.CompilerParams(use_tc_tiling_on_sc=False),
        scratch_types=[
            pltpu.VMEM((_NWIN, _W), jnp.int32),
            pltpu.VMEM((_NWIN, _W), jnp.int32),
            pltpu.VMEM((_NBUF, _W, _DH), jnp.float32),
            pltpu.VMEM_SHARED((_NP, _DH), jnp.float32),
            pltpu.SemaphoreType.DMA((_NBUF,)),
            pltpu.SemaphoreType.DMA((_NBUF,)),
        ],
    )
    return k(h2, src3, dst3)


def _tc_root_body(h_ref, wroot_ref, b_ref, out_ref):
    h = jnp.concatenate([h_ref[0], h_ref[1]], axis=1)
    res = jnp.dot(h, wroot_ref[...], precision=_HIGH,
                  preferred_element_type=jnp.float32) + b_ref[...]
    out_ref[0] = res[:, :_DH]
    out_ref[1] = res[:, _DH:]


def _tc_root(h2, wroot, b):
    return pl.pallas_call(
        _tc_root_body,
        grid=(_GRID,),
        in_specs=[
            pl.BlockSpec((_NC, _B, _DH), lambda i: (0, i, 0)),
            pl.BlockSpec((_D, _D), lambda i: (0, 0)),
            pl.BlockSpec((1, _D), lambda i: (0, 0)),
        ],
        out_specs=pl.BlockSpec((_NC, _B, _DH), lambda i: (0, i, 0)),
        out_shape=jax.ShapeDtypeStruct((_NC, _NP, _DH), jnp.float32),
    )(h2, wroot, b)


def _tc_layer_body(p_ref, r_ref, wrel_ref, out_ref):
    agg = jnp.concatenate([p_ref[0], p_ref[1]], axis=1)
    root = jnp.concatenate([r_ref[0], r_ref[1]], axis=1)
    acc = jnp.dot(agg, wrel_ref[...], precision=_HIGH,
                  preferred_element_type=jnp.float32)
    res = jnp.maximum(acc + root, 0.0)
    out_ref[0] = res[:, :_DH]
    out_ref[1] = res[:, _DH:]


def _tc_layer(p, r, wrel):
    return pl.pallas_call(
        _tc_layer_body,
        grid=(_GRID,),
        in_specs=[
            pl.BlockSpec((_NC, _B, _DH), lambda i: (0, i, 0)),
            pl.BlockSpec((_NC, _B, _DH), lambda i: (0, i, 0)),
            pl.BlockSpec((_D, _D), lambda i: (0, 0)),
        ],
        out_specs=pl.BlockSpec((_NC, _B, _DH), lambda i: (0, i, 0)),
        out_shape=jax.ShapeDtypeStruct((_NC, _NP, _DH), jnp.float32),
    )(p, r, wrel)


def _pool_body(h_ref, batch_ref, wfc1_ref, bfc1_ref, wfc2_ref, bfc2_ref,
               out_ref, pooled, counts):
    i = pl.program_id(0)

    @pl.when(i == 0)
    def _():
        pooled[...] = jnp.zeros_like(pooled)
        counts[...] = jnp.zeros_like(counts)

    h = jnp.concatenate([h_ref[0], h_ref[1]], axis=1)
    ids = lax.broadcasted_iota(jnp.int32, (_G, _B), 0)
    seg = (batch_ref[...] == ids).astype(jnp.float32)
    pooled[...] += jnp.dot(seg, h, precision=_HIGH,
                           preferred_element_type=jnp.float32)
    counts[...] += jnp.broadcast_to(jnp.sum(seg, axis=1)[:, None], (_G, _D))

    @pl.when(i == _GRID - 1)
    def _():
        pm = pooled[...] / jnp.maximum(counts[...], 1.0)
        o1 = jnp.dot(pm, wfc1_ref[...], precision=_HIGH,
                     preferred_element_type=jnp.float32) + bfc1_ref[...]
        out_ref[...] = jnp.dot(o1, wfc2_ref[...], precision=_HIGH,
                               preferred_element_type=jnp.float32) + bfc2_ref[...]


def _pool_mlp(h2, batch2, wfc1, bfc1, wfc2p, bfc2p):
    return pl.pallas_call(
        _pool_body,
        grid=(_GRID,),
        in_specs=[
            pl.BlockSpec((_NC, _B, _DH), lambda i: (0, i, 0)),
            pl.BlockSpec((1, _B), lambda i: (0, i)),
            pl.BlockSpec((_D, _D), lambda i: (0, 0)),
            pl.BlockSpec((1, _D), lambda i: (0, 0)),
            pl.BlockSpec((_D, _D), lambda i: (0, 0)),
            pl.BlockSpec((1, _D), lambda i: (0, 0)),
        ],
        out_specs=pl.BlockSpec((_G, _D), lambda i: (0, 0)),
        out_shape=jax.ShapeDtypeStruct((_G, _D), jnp.float32),
        scratch_shapes=[
            pltpu.VMEM((_G, _D), jnp.float32),
            pltpu.VMEM((_G, _D), jnp.float32),
        ],
    )(h2, batch2, wfc1, bfc1, wfc2p, bfc2p)


def kernel(x, edge_index, batch, W_rel1, b_rel1, W_root1, W_rel2, b_rel2,
           W_root2, W_rel3, b_rel3, W_root3, W_fc1, b_fc1, W_fc2, b_fc2):
    out_dim = W_fc2.shape[1]
    npad = _NP - _N
    epad = _EPAD - edge_index.shape[1]

    xp = jnp.concatenate([x, jnp.zeros((npad, _D), x.dtype)], axis=0)
    x2 = jnp.stack([xp[:, :_DH], xp[:, _DH:]])
    fill = (jnp.arange(epad, dtype=jnp.int32) % npad) + _N
    src3 = jnp.concatenate([edge_index[0], fill]).reshape(_NS, _NWIN, _W)
    dst3 = jnp.concatenate([edge_index[1], fill]).reshape(_NS, _NWIN, _W)
    batch2 = jnp.concatenate(
        [batch, jnp.full((npad,), _G, jnp.int32)]).reshape(1, _NP)

    h2 = x2
    for wrel, brel, wroot in ((W_rel1, b_rel1, W_root1),
                              (W_rel2, b_rel2, W_root2),
                              (W_rel3, b_rel3, W_root3)):
        p = _sc_aggregate(h2, src3, dst3)
        r = _tc_root(h2, wroot, brel.reshape(1, _D))
        h2 = _tc_layer(p, r, wrel)

    wfc2p = jnp.pad(W_fc2, ((0, 0), (0, _D - out_dim)))
    bfc2p = jnp.pad(b_fc2, (0, _D - out_dim)).reshape(1, _D)
    out = _pool_mlp(h2, batch2, W_fc1, b_fc1.reshape(1, _D), wfc2p, bfc2p)
    return out[:, :out_dim]
